# Initial kernel scaffold; baseline (speedup 1.0000x reference)
import jax
import jax.numpy as jnp
from jax.experimental import pallas as pl


def kernel(node_attr, edge_index, edge_attr, edge_sh, fc_w1, fc_b1, fc_w2, fc_b2):
    raise NotImplementedError("write your pallas kernel here")


# speedup vs baseline: 3.7267x; 3.7267x over previous
import functools

import jax
import jax.numpy as jnp
import numpy as np
from jax import lax
from jax.experimental import pallas as pl
from jax.experimental.pallas import tpu as pltpu
from jax.experimental.pallas import tpu_sc as plsc

NS = 16
NV = 4
N_NODES = 10000
N_EDGES = 160000
NF = 48
WN = 320
NORM = 1.0 / np.sqrt(NS)

NW = 32
EPW = N_EDGES // NW
CHUNK = 128
NCHUNK = EPW // CHUNK
TAIL = EPW - NCHUNK * CHUNK
GRP = 13
NGRP = NCHUNK // GRP
RPT = N_NODES // 16

TE = 1280
GRID = N_EDGES // TE

_PERM = np.concatenate([
    np.arange(NS * NS),
    np.array([NS * NS + i * NV + o for o in range(NV) for i in range(NS)]),
]).astype(np.int32)

_mesh = plsc.VectorSubcoreMesh(
    core_axis_name="c", subcore_axis_name="s", num_cores=2, num_subcores=16)
_sc_params = pltpu.CompilerParams(use_tc_tiling_on_sc=False)


def _gather_body(node_hbm, dst_hbm, h_hbm, idx_v, rows_v, sem):
    c = lax.axis_index("c")
    s = lax.axis_index("s")
    wid = s * 2 + c
    base = wid * EPW
    pltpu.sync_copy(dst_hbm.at[pl.ds(base, EPW)], idx_v)
    for g in range(NGRP):
        descs = []
        for j in range(GRP):
            k = g * GRP + j
            descs.append(pltpu.async_copy(
                node_hbm.at[idx_v.at[pl.ds(k * CHUNK, CHUNK)]],
                rows_v.at[pl.ds(k * CHUNK, CHUNK)], sem))
        for d in descs:
            d.wait()
    pltpu.async_copy(
        node_hbm.at[idx_v.at[pl.ds(NCHUNK * CHUNK, TAIL)]],
        rows_v.at[pl.ds(NCHUNK * CHUNK, TAIL)], sem).wait()
    pltpu.sync_copy(rows_v, h_hbm.at[pl.ds(base, EPW)])


_gather = functools.partial(
    pl.kernel,
    out_type=jax.ShapeDtypeStruct((N_EDGES, NS), jnp.float32),
    mesh=_mesh,
    compiler_params=_sc_params,
    scratch_types=[
        pltpu.VMEM((EPW,), jnp.int32),
        pltpu.VMEM((EPW, NS), jnp.float32),
        pltpu.SemaphoreType.DMA,
    ],
)(_gather_body)


def _dense_body(ea_ref, sh_ref, h_ref, w1t_ref, b1_ref, w2p_ref, b2p_ref,
                i16_ref, i9_ref, i32_ref, out_ref):
    f32 = jnp.float32
    dn_tr = (((1,), (1,)), ((), ()))
    hidT = lax.dot_general(w1t_ref[...], ea_ref[...], dn_tr,
                           preferred_element_type=f32)
    hidT = jnp.maximum(hidT + b1_ref[...], 0.0)
    wT = lax.dot_general(w2p_ref[...], hidT, (((1,), (0,)), ((), ())),
                         preferred_element_type=f32) + b2p_ref[...]
    hT = lax.dot_general(i16_ref[...], h_ref[...], dn_tr,
                         preferred_element_type=f32)
    shT = lax.dot_general(i9_ref[...], sh_ref[...], dn_tr,
                          preferred_element_type=f32)
    sh0 = shT[0:1, :]
    sh1 = shT[1:4, :]
    hxT = hT * sh0
    acc = jnp.zeros((NS, TE), f32)
    for i in range(NS):
        acc = acc + hxT[i:i + 1, :] * wT[i * NS:(i + 1) * NS, :]
    rows = [acc * NORM]
    for o in range(NV):
        lo = NS * NS + o * NS
        yo = jnp.sum(wT[lo:lo + NS, :] * hT, axis=0, keepdims=True)
        rows.append(yo * sh1 * NORM)
    rows.append(jnp.ones((1, TE), f32))
    rows.append(jnp.zeros((3, TE), f32))
    tpT = jnp.concatenate(rows, axis=0)
    out_ref[...] = lax.dot_general(tpT, i32_ref[...], (((0,), (0,)), ((), ())),
                                   preferred_element_type=f32)


def _dense_call(edge_attr, edge_sh, h, w1t, b1c, w2p, b2pc, i16, i9, i32):
    return pl.pallas_call(
        _dense_body,
        grid=(GRID,),
        in_specs=[
            pl.BlockSpec((TE, NF), lambda g: (g, 0)),
            pl.BlockSpec((TE, 9), lambda g: (g, 0)),
            pl.BlockSpec((TE, NS), lambda g: (g, 0)),
            pl.BlockSpec((NF, NF), lambda g: (0, 0)),
            pl.BlockSpec((NF, 1), lambda g: (0, 0)),
            pl.BlockSpec((WN, NF), lambda g: (0, 0)),
            pl.BlockSpec((WN, 1), lambda g: (0, 0)),
            pl.BlockSpec((NS, NS), lambda g: (0, 0)),
            pl.BlockSpec((9, 9), lambda g: (0, 0)),
            pl.BlockSpec((32, 32), lambda g: (0, 0)),
        ],
        out_specs=pl.BlockSpec((TE, 32), lambda g: (g, 0)),
        out_shape=jax.ShapeDtypeStruct((N_EDGES, 32), jnp.float32),
    )(edge_attr, edge_sh, h, w1t, b1c, w2p, b2pc, i16, i9, i32)


def _scatter_body(src_hbm, tp_hbm, zeros_hbm, part_hbm,
                  idx_v, upd_v, tidx_v, tupd_v, acc_sh, sem):
    c = lax.axis_index("c")
    s = lax.axis_index("s")
    base = c * (N_EDGES // 2) + s * EPW
    pltpu.sync_copy(zeros_hbm, acc_sh.at[pl.ds(s * RPT, RPT)])
    plsc.subcore_barrier()
    for g in range(NGRP):
        descs = []
        for j in range(GRP):
            k = g * GRP + j
            descs.append(pltpu.async_copy(
                src_hbm.at[pl.ds(base + k * CHUNK, CHUNK)], idx_v.at[j], sem))
            descs.append(pltpu.async_copy(
                tp_hbm.at[pl.ds(base + k * CHUNK, CHUNK)],
                upd_v.at[pl.ds(j * CHUNK, CHUNK)], sem))
        for d in descs:
            d.wait()
        for j in range(GRP):
            pltpu.sync_copy(upd_v.at[pl.ds(j * CHUNK, CHUNK)],
                            acc_sh.at[idx_v.at[j]], add=True)
    tb = base + NCHUNK * CHUNK
    pltpu.sync_copy(src_hbm.at[pl.ds(tb, TAIL)], tidx_v)
    pltpu.sync_copy(tp_hbm.at[pl.ds(tb, TAIL)], tupd_v)
    pltpu.sync_copy(tupd_v, acc_sh.at[tidx_v], add=True)
    plsc.subcore_barrier()
    pltpu.sync_copy(acc_sh.at[pl.ds(s * RPT, RPT)],
                    part_hbm.at[c, pl.ds(s * RPT, RPT)])


_scatter = functools.partial(
    pl.kernel,
    out_type=jax.ShapeDtypeStruct((2, N_NODES, 32), jnp.float32),
    mesh=_mesh,
    compiler_params=_sc_params,
    scratch_types=[
        pltpu.VMEM((GRP, CHUNK), jnp.int32),
        pltpu.VMEM((GRP * CHUNK, 32), jnp.float32),
        pltpu.VMEM((TAIL,), jnp.int32),
        pltpu.VMEM((TAIL, 32), jnp.float32),
        pltpu.VMEM_SHARED((N_NODES, 32), jnp.float32),
        pltpu.SemaphoreType.DMA,
    ],
)(_scatter_body)


def _fin_body(p_ref, out_ref):
    psum = p_ref[0] + p_ref[1]
    cnt = jnp.maximum(psum[:, 28:29], 1.0)
    out_ref[...] = psum / cnt


def _fin_call(partials):
    blk = N_NODES // 10
    return pl.pallas_call(
        _fin_body,
        grid=(10,),
        in_specs=[pl.BlockSpec((2, blk, 32), lambda g: (0, g, 0))],
        out_specs=pl.BlockSpec((blk, 32), lambda g: (g, 0)),
        out_shape=jax.ShapeDtypeStruct((N_NODES, 32), jnp.float32),
    )(partials)


def kernel(node_attr, edge_index, edge_attr, edge_sh, fc_w1, fc_b1, fc_w2, fc_b2):
    edge_src = edge_index[0]
    edge_dst = edge_index[1]
    w1t = fc_w1.T
    b1c = fc_b1.reshape(NF, 1)
    w2p = fc_w2.T[_PERM]
    b2pc = fc_b2[_PERM].reshape(WN, 1)
    i16 = jnp.eye(NS, dtype=jnp.float32)
    i9 = jnp.eye(9, dtype=jnp.float32)
    i32 = jnp.eye(32, dtype=jnp.float32)
    zeros = jnp.zeros((RPT, 32), jnp.float32)

    h = _gather(node_attr, edge_dst)
    tp = _dense_call(edge_attr, edge_sh, h, w1t, b1c, w2p, b2pc, i16, i9, i32)
    partials = _scatter(edge_src, tp, zeros)
    out = _fin_call(partials)
    return out[:, :28]
